# Initial kernel scaffold; baseline (speedup 1.0000x reference)
import jax
import jax.numpy as jnp
from jax.experimental import pallas as pl


def kernel(input_ids, embed_table):
    raise NotImplementedError("write your pallas kernel here")


# speedup vs baseline: 1.6394x; 1.6394x over previous
import functools

import jax
import jax.numpy as jnp
from jax import lax
from jax.experimental import pallas as pl
from jax.experimental.pallas import tpu as pltpu
from jax.experimental.pallas import tpu_sc as plsc

_NUM_WORKERS = 32


def _gather_kernel(n_rows, d, ids_hbm, table_hbm, out_hbm, idx_v, rows_v, gsem):
    b_per_w = n_rows // _NUM_WORKERS
    ch = rows_v.shape[0]
    n_chunks = b_per_w // ch
    wid = lax.axis_index("s") * 2 + lax.axis_index("c")
    base = wid * b_per_w
    pltpu.sync_copy(ids_hbm.at[pl.ds(base, b_per_w)], idx_v)

    def body(i, carry):
        pltpu.async_copy(
            table_hbm.at[idx_v.at[pl.ds(i * ch, ch)]], rows_v, gsem
        ).wait()
        pltpu.sync_copy(rows_v, out_hbm.at[pl.ds(base + i * ch, ch)])
        return carry

    lax.fori_loop(0, n_chunks, body, 0)


def kernel(input_ids, embed_table):
    b, s = input_ids.shape
    v, d = embed_table.shape
    n = b * s
    ids_flat = input_ids.reshape(n).astype(jnp.int32)
    ch = 32

    mesh = plsc.VectorSubcoreMesh(core_axis_name="c", subcore_axis_name="s")
    run = pl.kernel(
        functools.partial(_gather_kernel, n, d),
        mesh=mesh,
        out_type=jax.ShapeDtypeStruct((n, d), jnp.float32),
        scratch_types=[
            pltpu.VMEM((n // _NUM_WORKERS,), jnp.int32),
            pltpu.VMEM((ch, d), jnp.float32),
            pltpu.SemaphoreType.DMA,
        ],
    )
    out = run(ids_flat, embed_table)
    return out.reshape(b, s, d)


# speedup vs baseline: 1.7915x; 1.0928x over previous
import functools

import jax
import jax.numpy as jnp
from jax import lax
from jax.experimental import pallas as pl
from jax.experimental.pallas import tpu as pltpu
from jax.experimental.pallas import tpu_sc as plsc

_NUM_WORKERS = 32
_NBUF = 4
_CH = 8


def _gather_kernel(n_rows, d, ids_hbm, table_hbm, out_hbm, idx_v, buf, gsem, ssem):
    b_per_w = n_rows // _NUM_WORKERS
    n_chunks = b_per_w // _CH
    n_groups = n_chunks // _NBUF
    wid = lax.axis_index("s") * 2 + lax.axis_index("c")
    base = wid * b_per_w
    pltpu.sync_copy(ids_hbm.at[pl.ds(base, b_per_w)], idx_v)

    def gather(i, b2):
        return pltpu.make_async_copy(
            table_hbm.at[idx_v.at[pl.ds(i * _CH, _CH)]], buf.at[b2], gsem
        )

    def scatter(i, b2):
        return pltpu.make_async_copy(
            buf.at[b2], out_hbm.at[pl.ds(base + i * _CH, _CH)], ssem
        )

    for b2 in range(_NBUF):
        gather(b2, b2).start()

    def grp(g, carry):
        for b2 in range(_NBUF):
            i = g * _NBUF + b2
            gather(i, b2).wait()
            scatter(i, b2).start()

            @pl.when(g < n_groups - 1)
            def _():
                scatter(i, b2).wait()
                gather(i + _NBUF, b2).start()

        return carry

    lax.fori_loop(0, n_groups, grp, 0)
    for b2 in range(_NBUF):
        scatter(0, b2).wait()


def kernel(input_ids, embed_table):
    b, s = input_ids.shape
    v, d = embed_table.shape
    n = b * s
    ids_flat = input_ids.reshape(n).astype(jnp.int32)

    mesh = plsc.VectorSubcoreMesh(core_axis_name="c", subcore_axis_name="s")
    run = pl.kernel(
        functools.partial(_gather_kernel, n, d),
        mesh=mesh,
        out_type=jax.ShapeDtypeStruct((n, d), jnp.float32),
        scratch_types=[
            pltpu.VMEM((n // _NUM_WORKERS,), jnp.int32),
            pltpu.VMEM((_NBUF, _CH, d), jnp.float32),
            pltpu.SemaphoreType.DMA,
            pltpu.SemaphoreType.DMA,
        ],
    )
    out = run(ids_flat, embed_table)
    return out.reshape(b, s, d)


# speedup vs baseline: 1.7954x; 1.0022x over previous
import functools

import jax
import jax.numpy as jnp
from jax import lax
from jax.experimental import pallas as pl
from jax.experimental.pallas import tpu as pltpu
from jax.experimental.pallas import tpu_sc as plsc

_NUM_WORKERS = 32
_NBUF = 4
_CH = 8
_LAG = 2


def _gather_kernel(n_rows, d, ids_hbm, table_hbm, out_hbm, idx_v, buf, gsem, ssem):
    b_per_w = n_rows // _NUM_WORKERS
    n_chunks = b_per_w // _CH
    n_groups = n_chunks // _NBUF
    wid = lax.axis_index("s") * 2 + lax.axis_index("c")
    base = wid * b_per_w
    pltpu.sync_copy(ids_hbm.at[pl.ds(base, b_per_w)], idx_v)

    def gather(i, b2):
        return pltpu.make_async_copy(
            table_hbm.at[idx_v.at[pl.ds(i * _CH, _CH)]], buf.at[b2], gsem
        )

    def scatter(i, b2):
        return pltpu.make_async_copy(
            buf.at[b2], out_hbm.at[pl.ds(base + i * _CH, _CH)], ssem
        )

    for b2 in range(_NBUF):
        gather(b2, b2).start()

    def grp(g, carry):
        for b2 in range(_NBUF):
            i = g * _NBUF + b2
            gather(i, b2).wait()
            scatter(i, b2).start()
            b2p = (b2 - _LAG) % _NBUF
            cond = (g > 0) if b2 < _LAG else (g < n_groups - 1)

            @pl.when(cond)
            def _():
                scatter(i, b2p).wait()
                gather(i - _LAG + _NBUF, b2p).start()

        return carry

    lax.fori_loop(0, n_groups, grp, 0)
    for b2 in range(_NBUF):
        scatter(0, b2).wait()


def kernel(input_ids, embed_table):
    b, s = input_ids.shape
    v, d = embed_table.shape
    n = b * s
    ids_flat = input_ids.reshape(n).astype(jnp.int32)

    mesh = plsc.VectorSubcoreMesh(core_axis_name="c", subcore_axis_name="s")
    run = pl.kernel(
        functools.partial(_gather_kernel, n, d),
        mesh=mesh,
        out_type=jax.ShapeDtypeStruct((n, d), jnp.float32),
        scratch_types=[
            pltpu.VMEM((n // _NUM_WORKERS,), jnp.int32),
            pltpu.VMEM((_NBUF, _CH, d), jnp.float32),
            pltpu.SemaphoreType.DMA,
            pltpu.SemaphoreType.DMA,
        ],
    )
    out = run(ids_flat, embed_table)
    return out.reshape(b, s, d)


# speedup vs baseline: 1.8145x; 1.0106x over previous
import functools

import jax
import jax.numpy as jnp
from jax import lax
from jax.experimental import pallas as pl
from jax.experimental.pallas import tpu as pltpu
from jax.experimental.pallas import tpu_sc as plsc

_NUM_WORKERS = 32
_NBUF = 4
_CH = 8


def _gather_kernel(n_rows, d, ids_hbm, table_hbm, out_hbm,
                   idx_v, buf, region, gsem, csem, ssem, dsem):
    b_per_w = n_rows // _NUM_WORKERS
    n_chunks = b_per_w // _CH
    n_groups = n_chunks // _NBUF
    s = lax.axis_index("s")
    wid = s * 2 + lax.axis_index("c")
    base = wid * b_per_w
    pltpu.sync_copy(ids_hbm.at[pl.ds(base, b_per_w)], idx_v)
    myregion = region.at[s]

    def gather(i, b2):
        return pltpu.make_async_copy(
            table_hbm.at[idx_v.at[pl.ds(i * _CH, _CH)]], buf.at[b2], gsem
        )

    def direct(i):
        return pltpu.make_async_copy(
            buf.at[0], out_hbm.at[pl.ds(base + i * _CH, _CH)], dsem
        )

    def tospmem(b2):
        return pltpu.make_async_copy(buf.at[b2], myregion.at[b2 - 1], csem)

    def drain(i, b2):
        return pltpu.make_async_copy(
            myregion.at[b2 - 1], out_hbm.at[pl.ds(base + i * _CH, _CH)], ssem
        )

    for b2 in range(_NBUF):
        gather(b2, b2).start()

    def grp(g, carry):
        for b2 in range(_NBUF):
            i = g * _NBUF + b2
            gather(i, b2).wait()
            if b2 == 0:
                direct(i).start()

                @pl.when(g < n_groups - 1)
                def _():
                    direct(i).wait()
                    gather(i + _NBUF, 0).start()
            else:

                @pl.when(g > 0)
                def _():
                    drain(i, b2).wait()

                tospmem(b2).start()
                tospmem(b2).wait()
                drain(i, b2).start()

                @pl.when(g < n_groups - 1)
                def _():
                    gather(i + _NBUF, b2).start()

        return carry

    lax.fori_loop(0, n_groups, grp, 0)
    direct(0).wait()
    for b2 in range(1, _NBUF):
        drain(0, b2).wait()


def kernel(input_ids, embed_table):
    b, s = input_ids.shape
    v, d = embed_table.shape
    n = b * s
    ids_flat = input_ids.reshape(n).astype(jnp.int32)

    mesh = plsc.VectorSubcoreMesh(core_axis_name="c", subcore_axis_name="s")
    run = pl.kernel(
        functools.partial(_gather_kernel, n, d),
        mesh=mesh,
        out_type=jax.ShapeDtypeStruct((n, d), jnp.float32),
        scratch_types=[
            pltpu.VMEM((n // _NUM_WORKERS,), jnp.int32),
            pltpu.VMEM((_NBUF, _CH, d), jnp.float32),
            pltpu.MemorySpace.VMEM_SHARED((16, _NBUF - 1, _CH, d), jnp.float32),
            pltpu.SemaphoreType.DMA,
            pltpu.SemaphoreType.DMA,
            pltpu.SemaphoreType.DMA,
            pltpu.SemaphoreType.DMA,
        ],
    )
    out = run(ids_flat, embed_table)
    return out.reshape(b, s, d)
